# baseline (speedup 1.0000x reference)
import functools

import jax
import jax.numpy as jnp
from jax.experimental import pallas as pl
from jax.experimental.pallas import tpu as pltpu

B, H, W = 16, 32, 32
T = 64
K = 5
NIN = 6
LUT = 64
IN_SIZE = K * K
H0, H1 = 36, 6
N0, N1 = T * H0, T * H1
OH, OW = H - K + 1, W - K + 1
L = OH * OW

NG = 8
TG = T // NG
N0G, N1G = TG * H0, TG * H1

LP = 896
IPB = 4
LW = IPB * LP


def _pack_truth_table(lut_ref):
    s = (lut_ref[...] > 0).astype(jnp.int32)
    w = jnp.left_shift(jnp.ones((1, 32), jnp.int32),
                       jax.lax.broadcasted_iota(jnp.int32, (1, 32), 1))
    lo = jnp.sum(s[:, :32] * w, axis=1, keepdims=True)
    hi = jnp.sum(s[:, 32:] * w, axis=1, keepdims=True)
    return lo, hi


def _tt_extract(addr_i, lo, hi):
    use_hi = addr_i >= 32
    amt = addr_i & 31
    word = jnp.where(use_hi, hi, lo)
    return (jnp.right_shift(word, amt) & 1).astype(jnp.int8)


def _kernel_body(x_ref, idx0_ref, idx1_ref, idx2_ref,
                 lut0_ref, lut1_ref, lut2_ref, out_ref,
                 w0_s, bd1_s, w2_s, tt0lo_s, tt0hi_s, tt1lo_s, tt1hi_s):
    @pl.when(pl.program_id(1) == 0)
    def _build_tables():
        i_iota = jax.lax.broadcasted_iota(jnp.int32, (N0G, IN_SIZE), 1)
        w0 = jnp.zeros((N0G, IN_SIZE), jnp.int32)
        for j in range(NIN):
            w0 = w0 + jnp.where(idx0_ref[:, j:j + 1] == i_iota, 1 << j, 0)
        w0_s[...] = w0.astype(jnp.int8)

        r_iota = jax.lax.broadcasted_iota(jnp.int32, (N1G, N0G), 1)
        c_iota = jax.lax.broadcasted_iota(jnp.int32, (N1G, N0G), 0)
        t2 = r_iota // H0
        i_idx = r_iota - t2 * H0
        t_c = c_iota // H1
        val = jnp.zeros((N1G, N0G), jnp.int32)
        for j in range(NIN):
            val = val + jnp.where(idx1_ref[:, j:j + 1] == i_idx, 1 << j, 0)
        bd1_s[...] = jnp.where(t_c == t2, val, 0).astype(jnp.int8)

        r2 = jax.lax.broadcasted_iota(jnp.int32, (TG, N1G), 1)
        c2 = jax.lax.broadcasted_iota(jnp.int32, (TG, N1G), 0)
        t2b = r2 // H1
        i2 = r2 - t2b * H1
        val2 = jnp.zeros((TG, N1G), jnp.int32)
        for j in range(NIN):
            val2 = val2 + jnp.where(idx2_ref[:, j:j + 1] == i2, 1 << j, 0)
        w2_s[...] = jnp.where(c2 == t2b, val2, 0).astype(jnp.int8)

        lo0, hi0 = _pack_truth_table(lut0_ref)
        tt0lo_s[...], tt0hi_s[...] = lo0, hi0
        lo1, hi1 = _pack_truth_table(lut1_ref)
        tt1lo_s[...], tt1hi_s[...] = lo1, hi1

    bits_pt = jnp.concatenate(
        [(x_ref[im] > 0.0).astype(jnp.int8) for im in range(IPB)],
        axis=-1)

    addr0 = jnp.dot(w0_s[...], bits_pt[:IN_SIZE, :],
                    preferred_element_type=jnp.int32)
    b0 = _tt_extract(addr0, tt0lo_s[...], tt0hi_s[...])

    addr1 = jnp.dot(bd1_s[...], b0,
                    preferred_element_type=jnp.int32)
    b1 = _tt_extract(addr1, tt1lo_s[...], tt1hi_s[...])

    a2 = jnp.dot(w2_s[...], b1,
                 preferred_element_type=jnp.int32)

    m = [(a2 & (1 << j)) != 0 for j in range(NIN)]
    chunks = []
    for k in range(8):
        e = [lut2_ref[:, 8 * k + i:8 * k + i + 1] for i in range(8)]
        v0 = jnp.where(m[0], e[1], e[0])
        v1 = jnp.where(m[0], e[3], e[2])
        v2 = jnp.where(m[0], e[5], e[4])
        v3 = jnp.where(m[0], e[7], e[6])
        w0x = jnp.where(m[1], v1, v0)
        w1x = jnp.where(m[1], v3, v2)
        chunks.append(jnp.where(m[2], w1x, w0x))
    c0 = jnp.where(m[3], chunks[1], chunks[0])
    c1 = jnp.where(m[3], chunks[3], chunks[2])
    c2 = jnp.where(m[3], chunks[5], chunks[4])
    c3 = jnp.where(m[3], chunks[7], chunks[6])
    d0 = jnp.where(m[4], c1, c0)
    d1 = jnp.where(m[4], c3, c2)
    res = jnp.where(m[5], d1, d0)
    for im in range(IPB):
        out_ref[im] = res[:, im * LP:im * LP + L]


def _unfold_t(x):
    xi = x[:, 0]
    rows = [xi[:, ki:ki + OH, kj:kj + OW].reshape(B, 1, L)
            for ki in range(K) for kj in range(K)]
    rows.append(jnp.zeros((B, 32 - IN_SIZE, L), x.dtype))
    p = jnp.concatenate(rows, axis=1)
    return jnp.pad(p, ((0, 0), (0, 0), (0, LP - L)))


@functools.partial(jax.jit, static_argnums=())
def kernel(x, idx0, idx1, idx2, lut0, lut1, lut2):
    xs = _unfold_t(x)
    idx0r = idx0.reshape(N0, NIN)
    idx1r = idx1.reshape(N1, NIN)
    idx2r = idx2.reshape(T, NIN)
    lut0r = lut0.reshape(N0, LUT)
    lut1r = lut1.reshape(N1, LUT)
    lut2r = lut2.reshape(T, LUT)

    out = pl.pallas_call(
        _kernel_body,
        grid=(NG, B // IPB),
        in_specs=[
            pl.BlockSpec((IPB, 32, LP), lambda g, b: (b, 0, 0)),
            pl.BlockSpec((N0G, NIN), lambda g, b: (g, 0)),
            pl.BlockSpec((N1G, NIN), lambda g, b: (g, 0)),
            pl.BlockSpec((TG, NIN), lambda g, b: (g, 0)),
            pl.BlockSpec((N0G, LUT), lambda g, b: (g, 0)),
            pl.BlockSpec((N1G, LUT), lambda g, b: (g, 0)),
            pl.BlockSpec((TG, LUT), lambda g, b: (g, 0)),
        ],
        out_specs=pl.BlockSpec((IPB, TG, L), lambda g, b: (b, g, 0)),
        out_shape=jax.ShapeDtypeStruct((B, T, L), jnp.float32),
        scratch_shapes=[
            pltpu.VMEM((N0G, IN_SIZE), jnp.int8),
            pltpu.VMEM((N1G, N0G), jnp.int8),
            pltpu.VMEM((TG, N1G), jnp.int8),
            pltpu.VMEM((N0G, 1), jnp.int32),
            pltpu.VMEM((N0G, 1), jnp.int32),
            pltpu.VMEM((N1G, 1), jnp.int32),
            pltpu.VMEM((N1G, 1), jnp.int32),
        ],
    )(xs, idx0r, idx1r, idx2r, lut0r, lut1r, lut2r)
    return out.reshape(B, T, OH, OW)


# speedup vs baseline: 1.1769x; 1.1769x over previous
import functools

import jax
import jax.numpy as jnp
from jax.experimental import pallas as pl
from jax.experimental.pallas import tpu as pltpu

B, H, W = 16, 32, 32
T = 64
K = 5
NIN = 6
LUT = 64
IN_SIZE = K * K
H0, H1 = 36, 6
N0, N1 = T * H0, T * H1
OH, OW = H - K + 1, W - K + 1
L = OH * OW

NG = 8
TG = T // NG
N0G, N1G = TG * H0, TG * H1

LP = 784
IPB = 16
LW = IPB * LP


def _pack_truth_table(lut_ref):
    s = (lut_ref[...] > 0).astype(jnp.int32)
    w = jnp.left_shift(jnp.ones((1, 32), jnp.int32),
                       jax.lax.broadcasted_iota(jnp.int32, (1, 32), 1))
    lo = jnp.sum(s[:, :32] * w, axis=1, keepdims=True)
    hi = jnp.sum(s[:, 32:] * w, axis=1, keepdims=True)
    return lo, hi


def _tt_extract(addr_i, lo, hi):
    use_hi = addr_i >= 32
    amt = addr_i & 31
    word = jnp.where(use_hi, hi, lo)
    return (jnp.right_shift(word, amt) & 1).astype(jnp.int8)


def _kernel_body(x_ref, idx0_ref, idx1_ref, idx2_ref,
                 lut0_ref, lut1_ref, lut2_ref, out_ref,
                 w0_s, bd1_s, w2_s, tt0lo_s, tt0hi_s, tt1lo_s, tt1hi_s):
    @pl.when(pl.program_id(1) == 0)
    def _build_tables():
        i_iota = jax.lax.broadcasted_iota(jnp.int32, (N0G, IN_SIZE), 1)
        w0 = jnp.zeros((N0G, IN_SIZE), jnp.int32)
        for j in range(NIN):
            w0 = w0 + jnp.where(idx0_ref[:, j:j + 1] == i_iota, 1 << j, 0)
        w0_s[...] = w0.astype(jnp.int8)

        r_iota = jax.lax.broadcasted_iota(jnp.int32, (N1G, N0G), 1)
        c_iota = jax.lax.broadcasted_iota(jnp.int32, (N1G, N0G), 0)
        t2 = r_iota // H0
        i_idx = r_iota - t2 * H0
        t_c = c_iota // H1
        val = jnp.zeros((N1G, N0G), jnp.int32)
        for j in range(NIN):
            val = val + jnp.where(idx1_ref[:, j:j + 1] == i_idx, 1 << j, 0)
        bd1_s[...] = jnp.where(t_c == t2, val, 0).astype(jnp.int8)

        r2 = jax.lax.broadcasted_iota(jnp.int32, (TG, N1G), 1)
        c2 = jax.lax.broadcasted_iota(jnp.int32, (TG, N1G), 0)
        t2b = r2 // H1
        i2 = r2 - t2b * H1
        val2 = jnp.zeros((TG, N1G), jnp.int32)
        for j in range(NIN):
            val2 = val2 + jnp.where(idx2_ref[:, j:j + 1] == i2, 1 << j, 0)
        w2_s[...] = jnp.where(c2 == t2b, val2, 0).astype(jnp.int8)

        lo0, hi0 = _pack_truth_table(lut0_ref)
        tt0lo_s[...], tt0hi_s[...] = lo0, hi0
        lo1, hi1 = _pack_truth_table(lut1_ref)
        tt1lo_s[...], tt1hi_s[...] = lo1, hi1

    bits_pt = jnp.concatenate(
        [(x_ref[im] > 0.0).astype(jnp.int8) for im in range(IPB)],
        axis=-1)

    addr0 = jnp.dot(w0_s[...], bits_pt[:IN_SIZE, :],
                    preferred_element_type=jnp.int32)
    b0 = _tt_extract(addr0, tt0lo_s[...], tt0hi_s[...])

    addr1 = jnp.dot(bd1_s[...], b0,
                    preferred_element_type=jnp.int32)
    b1 = _tt_extract(addr1, tt1lo_s[...], tt1hi_s[...])

    a2 = jnp.dot(w2_s[...], b1,
                 preferred_element_type=jnp.int32)

    m = [(a2 & (1 << j)) != 0 for j in range(NIN)]
    chunks = []
    for k in range(8):
        e = [lut2_ref[:, 8 * k + i:8 * k + i + 1] for i in range(8)]
        v0 = jnp.where(m[0], e[1], e[0])
        v1 = jnp.where(m[0], e[3], e[2])
        v2 = jnp.where(m[0], e[5], e[4])
        v3 = jnp.where(m[0], e[7], e[6])
        w0x = jnp.where(m[1], v1, v0)
        w1x = jnp.where(m[1], v3, v2)
        chunks.append(jnp.where(m[2], w1x, w0x))
    c0 = jnp.where(m[3], chunks[1], chunks[0])
    c1 = jnp.where(m[3], chunks[3], chunks[2])
    c2 = jnp.where(m[3], chunks[5], chunks[4])
    c3 = jnp.where(m[3], chunks[7], chunks[6])
    d0 = jnp.where(m[4], c1, c0)
    d1 = jnp.where(m[4], c3, c2)
    res = jnp.where(m[5], d1, d0)
    for im in range(IPB):
        out_ref[im] = res[:, im * LP:im * LP + L]


def _unfold_t(x):
    xi = x[:, 0]
    rows = [xi[:, ki:ki + OH, kj:kj + OW].reshape(B, 1, L)
            for ki in range(K) for kj in range(K)]
    rows.append(jnp.zeros((B, 32 - IN_SIZE, L), x.dtype))
    return jnp.concatenate(rows, axis=1)


@functools.partial(jax.jit, static_argnums=())
def kernel(x, idx0, idx1, idx2, lut0, lut1, lut2):
    xs = _unfold_t(x)
    idx0r = idx0.reshape(N0, NIN)
    idx1r = idx1.reshape(N1, NIN)
    idx2r = idx2.reshape(T, NIN)
    lut0r = lut0.reshape(N0, LUT)
    lut1r = lut1.reshape(N1, LUT)
    lut2r = lut2.reshape(T, LUT)

    out = pl.pallas_call(
        _kernel_body,
        grid=(NG, B // IPB),
        in_specs=[
            pl.BlockSpec((IPB, 32, LP), lambda g, b: (b, 0, 0)),
            pl.BlockSpec((N0G, NIN), lambda g, b: (g, 0)),
            pl.BlockSpec((N1G, NIN), lambda g, b: (g, 0)),
            pl.BlockSpec((TG, NIN), lambda g, b: (g, 0)),
            pl.BlockSpec((N0G, LUT), lambda g, b: (g, 0)),
            pl.BlockSpec((N1G, LUT), lambda g, b: (g, 0)),
            pl.BlockSpec((TG, LUT), lambda g, b: (g, 0)),
        ],
        out_specs=pl.BlockSpec((IPB, TG, L), lambda g, b: (b, g, 0)),
        out_shape=jax.ShapeDtypeStruct((B, T, L), jnp.float32),
        scratch_shapes=[
            pltpu.VMEM((N0G, IN_SIZE), jnp.int8),
            pltpu.VMEM((N1G, N0G), jnp.int8),
            pltpu.VMEM((TG, N1G), jnp.int8),
            pltpu.VMEM((N0G, 1), jnp.int32),
            pltpu.VMEM((N0G, 1), jnp.int32),
            pltpu.VMEM((N1G, 1), jnp.int32),
            pltpu.VMEM((N1G, 1), jnp.int32),
        ],
    )(xs, idx0r, idx1r, idx2r, lut0r, lut1r, lut2r)
    return out.reshape(B, T, OH, OW)


# speedup vs baseline: 1.1794x; 1.0022x over previous
import functools

import jax
import jax.numpy as jnp
from jax.experimental import pallas as pl
from jax.experimental.pallas import tpu as pltpu

B, H, W = 16, 32, 32
T = 64
K = 5
NIN = 6
LUT = 64
IN_SIZE = K * K
H0, H1 = 36, 6
N0, N1 = T * H0, T * H1
OH, OW = H - K + 1, W - K + 1
L = OH * OW

NG = 4
TG = T // NG
N0G, N1G = TG * H0, TG * H1

LP = 784
IPB = 16
LW = IPB * LP


def _pack_truth_table(lut_ref):
    s = (lut_ref[...] > 0).astype(jnp.int32)
    w = jnp.left_shift(jnp.ones((1, 32), jnp.int32),
                       jax.lax.broadcasted_iota(jnp.int32, (1, 32), 1))
    lo = jnp.sum(s[:, :32] * w, axis=1, keepdims=True)
    hi = jnp.sum(s[:, 32:] * w, axis=1, keepdims=True)
    return lo, hi


def _tt_extract(addr_i, lo, hi):
    use_hi = addr_i >= 32
    amt = addr_i & 31
    word = jnp.where(use_hi, hi, lo)
    return (jnp.right_shift(word, amt) & 1).astype(jnp.int8)


def _kernel_body(x_ref, idx0_ref, idx1_ref, idx2_ref,
                 lut0_ref, lut1_ref, lut2_ref, out_ref,
                 w0_s, bd1_s, w2_s, tt0lo_s, tt0hi_s, tt1lo_s, tt1hi_s):
    @pl.when(pl.program_id(1) == 0)
    def _build_tables():
        i_iota = jax.lax.broadcasted_iota(jnp.int32, (N0G, IN_SIZE), 1)
        w0 = jnp.zeros((N0G, IN_SIZE), jnp.int32)
        for j in range(NIN):
            w0 = w0 + jnp.where(idx0_ref[:, j:j + 1] == i_iota, 1 << j, 0)
        w0_s[...] = w0.astype(jnp.int8)

        r_iota = jax.lax.broadcasted_iota(jnp.int32, (N1G, N0G), 1)
        c_iota = jax.lax.broadcasted_iota(jnp.int32, (N1G, N0G), 0)
        t2 = r_iota // H0
        i_idx = r_iota - t2 * H0
        t_c = c_iota // H1
        val = jnp.zeros((N1G, N0G), jnp.int32)
        for j in range(NIN):
            val = val + jnp.where(idx1_ref[:, j:j + 1] == i_idx, 1 << j, 0)
        bd1_s[...] = jnp.where(t_c == t2, val, 0).astype(jnp.int8)

        r2 = jax.lax.broadcasted_iota(jnp.int32, (TG, N1G), 1)
        c2 = jax.lax.broadcasted_iota(jnp.int32, (TG, N1G), 0)
        t2b = r2 // H1
        i2 = r2 - t2b * H1
        val2 = jnp.zeros((TG, N1G), jnp.int32)
        for j in range(NIN):
            val2 = val2 + jnp.where(idx2_ref[:, j:j + 1] == i2, 1 << j, 0)
        w2_s[...] = jnp.where(c2 == t2b, val2, 0).astype(jnp.int8)

        lo0, hi0 = _pack_truth_table(lut0_ref)
        tt0lo_s[...], tt0hi_s[...] = lo0, hi0
        lo1, hi1 = _pack_truth_table(lut1_ref)
        tt1lo_s[...], tt1hi_s[...] = lo1, hi1

    bits_pt = jnp.concatenate(
        [(x_ref[im] > 0.0).astype(jnp.int8) for im in range(IPB)],
        axis=-1)

    addr0 = jnp.dot(w0_s[...], bits_pt[:IN_SIZE, :],
                    preferred_element_type=jnp.int32)
    b0 = _tt_extract(addr0, tt0lo_s[...], tt0hi_s[...])

    addr1 = jnp.dot(bd1_s[...], b0,
                    preferred_element_type=jnp.int32)
    b1 = _tt_extract(addr1, tt1lo_s[...], tt1hi_s[...])

    a2 = jnp.dot(w2_s[...], b1,
                 preferred_element_type=jnp.int32)

    m = [(a2 & (1 << j)) != 0 for j in range(NIN)]
    chunks = []
    for k in range(8):
        e = [lut2_ref[:, 8 * k + i:8 * k + i + 1] for i in range(8)]
        v0 = jnp.where(m[0], e[1], e[0])
        v1 = jnp.where(m[0], e[3], e[2])
        v2 = jnp.where(m[0], e[5], e[4])
        v3 = jnp.where(m[0], e[7], e[6])
        w0x = jnp.where(m[1], v1, v0)
        w1x = jnp.where(m[1], v3, v2)
        chunks.append(jnp.where(m[2], w1x, w0x))
    c0 = jnp.where(m[3], chunks[1], chunks[0])
    c1 = jnp.where(m[3], chunks[3], chunks[2])
    c2 = jnp.where(m[3], chunks[5], chunks[4])
    c3 = jnp.where(m[3], chunks[7], chunks[6])
    d0 = jnp.where(m[4], c1, c0)
    d1 = jnp.where(m[4], c3, c2)
    res = jnp.where(m[5], d1, d0)
    for im in range(IPB):
        out_ref[im] = res[:, im * LP:im * LP + L]


def _unfold_t(x):
    xi = x[:, 0]
    rows = [xi[:, ki:ki + OH, kj:kj + OW].reshape(B, 1, L)
            for ki in range(K) for kj in range(K)]
    rows.append(jnp.zeros((B, 32 - IN_SIZE, L), x.dtype))
    return jnp.concatenate(rows, axis=1)


@functools.partial(jax.jit, static_argnums=())
def kernel(x, idx0, idx1, idx2, lut0, lut1, lut2):
    xs = _unfold_t(x)
    idx0r = idx0.reshape(N0, NIN)
    idx1r = idx1.reshape(N1, NIN)
    idx2r = idx2.reshape(T, NIN)
    lut0r = lut0.reshape(N0, LUT)
    lut1r = lut1.reshape(N1, LUT)
    lut2r = lut2.reshape(T, LUT)

    out = pl.pallas_call(
        _kernel_body,
        grid=(NG, B // IPB),
        in_specs=[
            pl.BlockSpec((IPB, 32, LP), lambda g, b: (b, 0, 0)),
            pl.BlockSpec((N0G, NIN), lambda g, b: (g, 0)),
            pl.BlockSpec((N1G, NIN), lambda g, b: (g, 0)),
            pl.BlockSpec((TG, NIN), lambda g, b: (g, 0)),
            pl.BlockSpec((N0G, LUT), lambda g, b: (g, 0)),
            pl.BlockSpec((N1G, LUT), lambda g, b: (g, 0)),
            pl.BlockSpec((TG, LUT), lambda g, b: (g, 0)),
        ],
        out_specs=pl.BlockSpec((IPB, TG, L), lambda g, b: (b, g, 0)),
        out_shape=jax.ShapeDtypeStruct((B, T, L), jnp.float32),
        scratch_shapes=[
            pltpu.VMEM((N0G, IN_SIZE), jnp.int8),
            pltpu.VMEM((N1G, N0G), jnp.int8),
            pltpu.VMEM((TG, N1G), jnp.int8),
            pltpu.VMEM((N0G, 1), jnp.int32),
            pltpu.VMEM((N0G, 1), jnp.int32),
            pltpu.VMEM((N1G, 1), jnp.int32),
            pltpu.VMEM((N1G, 1), jnp.int32),
        ],
    )(xs, idx0r, idx1r, idx2r, lut0r, lut1r, lut2r)
    return out.reshape(B, T, OH, OW)


# speedup vs baseline: 1.2045x; 1.0213x over previous
import functools

_MAGIC_F = float(3 * 2 ** 22)

import jax
import jax.numpy as jnp
from jax.experimental import pallas as pl
from jax.experimental.pallas import tpu as pltpu

B, H, W = 16, 32, 32
T = 64
K = 5
NIN = 6
LUT = 64
IN_SIZE = K * K
H0, H1 = 36, 6
N0, N1 = T * H0, T * H1
OH, OW = H - K + 1, W - K + 1
L = OH * OW

NG = 4
TG = T // NG
N0G, N1G = TG * H0, TG * H1

LP = 784
IPB = 16
LW = IPB * LP


def _pack_truth_table(lut_ref):
    s = (lut_ref[...] > 0).astype(jnp.int32)
    w = jnp.left_shift(jnp.ones((1, 32), jnp.int32),
                       jax.lax.broadcasted_iota(jnp.int32, (1, 32), 1))
    lo = jnp.sum(s[:, :32] * w, axis=1, keepdims=True)
    hi = jnp.sum(s[:, 32:] * w, axis=1, keepdims=True)
    return lo, hi


def _tt_extract(addr_f, lo, hi):
    u = jax.lax.bitcast_convert_type(addr_f + _MAGIC_F, jnp.int32)
    word = jnp.where(addr_f < 0.0, hi, lo)
    return (jnp.right_shift(word, u & 31) & 1).astype(jnp.int8)


def _kernel_body(x_ref, idx0_ref, idx1_ref, idx2_ref,
                 lut0_ref, lut1_ref, lut2_ref, out_ref,
                 w0_s, bd1_s, w2_s, tt0lo_s, tt0hi_s, tt1lo_s, tt1hi_s):
    @pl.when(pl.program_id(1) == 0)
    def _build_tables():
        i_iota = jax.lax.broadcasted_iota(jnp.int32, (N0G, IN_SIZE), 1)
        w0 = jnp.zeros((N0G, IN_SIZE), jnp.int32)
        for j in range(NIN):
            w0 = w0 + jnp.where(idx0_ref[:, j:j + 1] == i_iota, -32 if j == 5 else 1 << j, 0)
        w0_s[...] = w0.astype(jnp.bfloat16)

        r_iota = jax.lax.broadcasted_iota(jnp.int32, (N1G, N0G), 1)
        c_iota = jax.lax.broadcasted_iota(jnp.int32, (N1G, N0G), 0)
        t2 = r_iota // H0
        i_idx = r_iota - t2 * H0
        t_c = c_iota // H1
        val = jnp.zeros((N1G, N0G), jnp.int32)
        for j in range(NIN):
            val = val + jnp.where(idx1_ref[:, j:j + 1] == i_idx, -32 if j == 5 else 1 << j, 0)
        bd1_s[...] = jnp.where(t_c == t2, val, 0).astype(jnp.bfloat16)

        r2 = jax.lax.broadcasted_iota(jnp.int32, (TG, N1G), 1)
        c2 = jax.lax.broadcasted_iota(jnp.int32, (TG, N1G), 0)
        t2b = r2 // H1
        i2 = r2 - t2b * H1
        val2 = jnp.zeros((TG, N1G), jnp.int32)
        for j in range(NIN):
            val2 = val2 + jnp.where(idx2_ref[:, j:j + 1] == i2, -32 if j == 5 else 1 << j, 0)
        w2_s[...] = jnp.where(c2 == t2b, val2, 0).astype(jnp.bfloat16)

        lo0, hi0 = _pack_truth_table(lut0_ref)
        tt0lo_s[...], tt0hi_s[...] = lo0, hi0
        lo1, hi1 = _pack_truth_table(lut1_ref)
        tt1lo_s[...], tt1hi_s[...] = lo1, hi1

    bits_pt = jnp.concatenate(
        [(x_ref[im] > 0.0).astype(jnp.bfloat16) for im in range(IPB)],
        axis=-1)

    addr0 = jnp.dot(w0_s[...], bits_pt[:IN_SIZE, :],
                    preferred_element_type=jnp.float32)
    b0 = _tt_extract(addr0, tt0lo_s[...], tt0hi_s[...])

    addr1 = jnp.dot(bd1_s[...], b0,
                    preferred_element_type=jnp.float32)
    b1 = _tt_extract(addr1, tt1lo_s[...], tt1hi_s[...])

    a2f = jnp.dot(w2_s[...], b1,
                  preferred_element_type=jnp.float32)
    a2 = jax.lax.bitcast_convert_type(a2f + _MAGIC_F, jnp.int32)

    m = [(a2 & (1 << j)) != 0 for j in range(NIN)]
    chunks = []
    for k in range(8):
        e = [lut2_ref[:, 8 * k + i:8 * k + i + 1] for i in range(8)]
        v0 = jnp.where(m[0], e[1], e[0])
        v1 = jnp.where(m[0], e[3], e[2])
        v2 = jnp.where(m[0], e[5], e[4])
        v3 = jnp.where(m[0], e[7], e[6])
        w0x = jnp.where(m[1], v1, v0)
        w1x = jnp.where(m[1], v3, v2)
        chunks.append(jnp.where(m[2], w1x, w0x))
    c0 = jnp.where(m[3], chunks[1], chunks[0])
    c1 = jnp.where(m[3], chunks[3], chunks[2])
    c2 = jnp.where(m[3], chunks[5], chunks[4])
    c3 = jnp.where(m[3], chunks[7], chunks[6])
    d0 = jnp.where(m[4], c1, c0)
    d1 = jnp.where(m[4], c3, c2)
    res = jnp.where(m[5], d1, d0)
    for im in range(IPB):
        out_ref[im] = res[:, im * LP:im * LP + L]


def _unfold_t(x):
    xi = x[:, 0]
    rows = [xi[:, ki:ki + OH, kj:kj + OW].reshape(B, 1, L)
            for ki in range(K) for kj in range(K)]
    rows.append(jnp.zeros((B, 32 - IN_SIZE, L), x.dtype))
    return jnp.concatenate(rows, axis=1)


@functools.partial(jax.jit, static_argnums=())
def kernel(x, idx0, idx1, idx2, lut0, lut1, lut2):
    xs = _unfold_t(x)
    idx0r = idx0.reshape(N0, NIN)
    idx1r = idx1.reshape(N1, NIN)
    idx2r = idx2.reshape(T, NIN)
    lut0r = lut0.reshape(N0, LUT)
    lut1r = lut1.reshape(N1, LUT)
    lut2r = lut2.reshape(T, LUT)

    out = pl.pallas_call(
        _kernel_body,
        grid=(NG, B // IPB),
        in_specs=[
            pl.BlockSpec((IPB, 32, LP), lambda g, b: (b, 0, 0)),
            pl.BlockSpec((N0G, NIN), lambda g, b: (g, 0)),
            pl.BlockSpec((N1G, NIN), lambda g, b: (g, 0)),
            pl.BlockSpec((TG, NIN), lambda g, b: (g, 0)),
            pl.BlockSpec((N0G, LUT), lambda g, b: (g, 0)),
            pl.BlockSpec((N1G, LUT), lambda g, b: (g, 0)),
            pl.BlockSpec((TG, LUT), lambda g, b: (g, 0)),
        ],
        out_specs=pl.BlockSpec((IPB, TG, L), lambda g, b: (b, g, 0)),
        out_shape=jax.ShapeDtypeStruct((B, T, L), jnp.float32),
        scratch_shapes=[
            pltpu.VMEM((N0G, IN_SIZE), jnp.bfloat16),
            pltpu.VMEM((N1G, N0G), jnp.bfloat16),
            pltpu.VMEM((TG, N1G), jnp.bfloat16),
            pltpu.VMEM((N0G, 1), jnp.int32),
            pltpu.VMEM((N0G, 1), jnp.int32),
            pltpu.VMEM((N1G, 1), jnp.int32),
            pltpu.VMEM((N1G, 1), jnp.int32),
        ],
    )(xs, idx0r, idx1r, idx2r, lut0r, lut1r, lut2r)
    return out.reshape(B, T, OH, OW)
